# Initial kernel scaffold; baseline (speedup 1.0000x reference)
import functools

import jax
import jax.numpy as jnp
from jax import lax
from jax.experimental import pallas as pl
from jax.experimental.pallas import tpu as pltpu
from jax.experimental.pallas import tpu_sc as plsc

V, D, O = 1_000_000, 64, 256
B, L = 4096, 200
NC, NS = 2, 16
NW = NC * NS
NB = B // NW
C0, C1 = 104, 96


def _pool_body(tokens_hbm, table_hbm, pooled_hbm, tok_v, buf0, buf1, out_v,
               sem0, sem1):
    wid = lax.axis_index("s") * NC + lax.axis_index("c")
    base = wid * NB
    pltpu.sync_copy(tokens_hbm.at[pl.ds(base, NB)], tok_v)

    def fire(b, buf, sem):
        pltpu.async_copy(table_hbm.at[tok_v.at[b, pl.ds(0, C0)]],
                         buf.at[pl.ds(0, C0)], sem)
        pltpu.async_copy(table_hbm.at[tok_v.at[b, pl.ds(C0, C1)]],
                         buf.at[pl.ds(C0, C1)], sem)

    def wait(buf, sem):
        pltpu.make_async_copy(table_hbm.at[pl.ds(0, L)], buf, sem).wait()

    lanes = lax.iota(jnp.int32, 16)
    ones = jnp.ones((16,), jnp.float32)
    zeros = jnp.zeros((16,), jnp.float32)

    def process(b, buf):
        cnt = zeros
        for j in range(12):
            t = tok_v[b, pl.ds(j * 16, 16)]
            cnt = cnt + jnp.where(t != 0, ones, zeros)
        t = tok_v[b, pl.ds(L - 16, 16)]
        cnt = cnt + jnp.where((t != 0) & (lanes >= 8), ones, zeros)
        n = jnp.sum(cnt)
        nvec = jnp.maximum(jnp.broadcast_to(n, (16,)), ones)
        inv = ones / nvec

        def acc_body(i8, accs):
            t0 = i8 * 8
            for dt in range(8):
                accs = tuple(a + buf[t0 + dt, pl.ds(k * 16, 16)]
                             for k, a in enumerate(accs))
            return accs

        accs = lax.fori_loop(0, L // 8, acc_body, (zeros, zeros, zeros, zeros))
        for k in range(4):
            out_v[b, pl.ds(k * 16, 16)] = accs[k] * inv

    fire(0, buf0, sem0)
    fire(1, buf1, sem1)

    def loop_body(i, carry):
        b0 = 2 * i
        wait(buf0, sem0)
        process(b0, buf0)

        @pl.when(i < NB // 2 - 1)
        def _():
            fire(b0 + 2, buf0, sem0)

        wait(buf1, sem1)
        process(b0 + 1, buf1)

        @pl.when(i < NB // 2 - 1)
        def _():
            fire(b0 + 3, buf1, sem1)

        return carry

    lax.fori_loop(0, NB // 2, loop_body, 0)
    pltpu.sync_copy(out_v, pooled_hbm.at[pl.ds(base, NB)])


_pool = functools.partial(
    pl.kernel,
    mesh=plsc.VectorSubcoreMesh(core_axis_name="c", subcore_axis_name="s"),
    out_type=jax.ShapeDtypeStruct((B, D), jnp.float32),
    scratch_types=[
        pltpu.VMEM((NB, L), jnp.int32),
        pltpu.VMEM((L, D), jnp.float32),
        pltpu.VMEM((L, D), jnp.float32),
        pltpu.VMEM((NB, D), jnp.float32),
        pltpu.SemaphoreType.DMA,
        pltpu.SemaphoreType.DMA,
    ],
)(_pool_body)


def _mlp_body(x_ref, w1_ref, b1_ref, w2_ref, b2_ref, o_ref):
    x = x_ref[...]
    h = jnp.dot(x, w1_ref[...], preferred_element_type=jnp.float32)
    h = jnp.maximum(h + b1_ref[...], 0.0)
    p = jnp.dot(h, w2_ref[...], preferred_element_type=jnp.float32)
    p = p + b2_ref[...]
    norm = jnp.sqrt(jnp.sum(p * p, axis=-1, keepdims=True))
    o_ref[...] = p / jnp.maximum(norm, 1e-8)


BLK = 512


def _mlp(pooled, W1, b1, W2, b2):
    return pl.pallas_call(
        _mlp_body,
        out_shape=jax.ShapeDtypeStruct((B, O), jnp.float32),
        grid=(B // BLK,),
        in_specs=[
            pl.BlockSpec((BLK, D), lambda i: (i, 0)),
            pl.BlockSpec((D, O), lambda i: (0, 0)),
            pl.BlockSpec((1, O), lambda i: (0, 0)),
            pl.BlockSpec((O, O), lambda i: (0, 0)),
            pl.BlockSpec((1, O), lambda i: (0, 0)),
        ],
        out_specs=pl.BlockSpec((BLK, O), lambda i: (i, 0)),
    )(pooled, W1, b1, W2, b2)


def kernel(tokens, table, W1, b1, W2, b2):
    pooled = _pool(tokens, table)
    return _mlp(pooled, W1, b1.reshape(1, O), W2, b2.reshape(1, O))


# speedup vs baseline: 1.0627x; 1.0627x over previous
import functools

import jax
import jax.numpy as jnp
from jax import lax
from jax.experimental import pallas as pl
from jax.experimental.pallas import tpu as pltpu
from jax.experimental.pallas import tpu_sc as plsc

V, D, O = 1_000_000, 64, 256
B, L = 4096, 200
NC, NS = 2, 16
NW = NC * NS
NB = B // NW
C0, C1 = 104, 96


def _pool_body(tokens_hbm, table_hbm, pooled_hbm, tok_v, buf0, buf1, out_v,
               sem0, sem1):
    wid = lax.axis_index("s") * NC + lax.axis_index("c")
    base = wid * NB
    pltpu.sync_copy(tokens_hbm.at[pl.ds(base * L, NB * L)], tok_v)

    def fire(b, buf, sem):
        pltpu.async_copy(table_hbm.at[tok_v.at[pl.ds(b * L, C0)]],
                         buf.at[pl.ds(0, C0)], sem)
        pltpu.async_copy(table_hbm.at[tok_v.at[pl.ds(b * L + C0, C1)]],
                         buf.at[pl.ds(C0, C1)], sem)

    def wait(buf, sem):
        pltpu.make_async_copy(table_hbm.at[pl.ds(0, L)], buf, sem).wait()

    zeros = jnp.zeros((16,), jnp.float32)

    def process(b, buf):
        def acc_body(i8, accs):
            t0 = i8 * 8
            for dt in range(8):
                accs = tuple(a + buf[t0 + dt, pl.ds(k * 16, 16)]
                             for k, a in enumerate(accs))
            return accs

        accs = lax.fori_loop(0, L // 8, acc_body, (zeros, zeros, zeros, zeros))
        for k in range(4):
            out_v[pl.ds(b * D + k * 16, 16)] = accs[k]

    fire(0, buf0, sem0)
    fire(1, buf1, sem1)

    def loop_body(i, carry):
        b0 = 2 * i
        wait(buf0, sem0)
        process(b0, buf0)

        @pl.when(i < NB // 2 - 1)
        def _():
            fire(b0 + 2, buf0, sem0)

        wait(buf1, sem1)
        process(b0 + 1, buf1)

        @pl.when(i < NB // 2 - 1)
        def _():
            fire(b0 + 3, buf1, sem1)

        return carry

    lax.fori_loop(0, NB // 2, loop_body, 0)
    pltpu.sync_copy(out_v, pooled_hbm.at[pl.ds(base * D, NB * D)])


_pool = functools.partial(
    pl.kernel,
    mesh=plsc.VectorSubcoreMesh(core_axis_name="c", subcore_axis_name="s"),
    compiler_params=pltpu.CompilerParams(use_tc_tiling_on_sc=False),
    out_type=jax.ShapeDtypeStruct((B * D,), jnp.float32),
    scratch_types=[
        pltpu.VMEM((NB * L,), jnp.int32),
        pltpu.VMEM((L, D), jnp.float32),
        pltpu.VMEM((L, D), jnp.float32),
        pltpu.VMEM((NB * D,), jnp.float32),
        pltpu.SemaphoreType.DMA,
        pltpu.SemaphoreType.DMA,
    ],
)(_pool_body)


def _mlp_body(x_ref, tok_ref, w1_ref, b1_ref, w2_ref, b2_ref, o_ref):
    cnt = jnp.sum((tok_ref[...] != 0).astype(jnp.float32), axis=1,
                  keepdims=True)
    x = x_ref[...] / jnp.maximum(cnt, 1.0)
    h = jnp.dot(x, w1_ref[...], preferred_element_type=jnp.float32)
    h = jnp.maximum(h + b1_ref[...], 0.0)
    p = jnp.dot(h, w2_ref[...], preferred_element_type=jnp.float32)
    p = p + b2_ref[...]
    norm = jnp.sqrt(jnp.sum(p * p, axis=-1, keepdims=True))
    o_ref[...] = p / jnp.maximum(norm, 1e-8)


BLK = 512


def _mlp(summed, tokens, W1, b1, W2, b2):
    return pl.pallas_call(
        _mlp_body,
        out_shape=jax.ShapeDtypeStruct((B, O), jnp.float32),
        grid=(B // BLK,),
        in_specs=[
            pl.BlockSpec((BLK, D), lambda i: (i, 0)),
            pl.BlockSpec((BLK, L), lambda i: (i, 0)),
            pl.BlockSpec((D, O), lambda i: (0, 0)),
            pl.BlockSpec((1, O), lambda i: (0, 0)),
            pl.BlockSpec((O, O), lambda i: (0, 0)),
            pl.BlockSpec((1, O), lambda i: (0, 0)),
        ],
        out_specs=pl.BlockSpec((BLK, O), lambda i: (i, 0)),
    )(summed, tokens, W1, b1, W2, b2)


def kernel(tokens, table, W1, b1, W2, b2):
    summed = _pool(tokens.reshape(-1), table).reshape(B, D)
    return _mlp(summed, tokens, W1, b1.reshape(1, O), W2, b2.reshape(1, O))


# speedup vs baseline: 1.0658x; 1.0029x over previous
import functools

import jax
import jax.numpy as jnp
from jax import lax
from jax.experimental import pallas as pl
from jax.experimental.pallas import tpu as pltpu
from jax.experimental.pallas import tpu_sc as plsc

V, D, O = 1_000_000, 64, 256
B, L = 4096, 200
NC, NS = 2, 16
NW = NC * NS
NB = B // NW
C0, C1 = 104, 96


def _pool_body(tokens_hbm, table_hbm, pooled_hbm, tok_v, buf0, buf1, out_v,
               sem0, sem1):
    wid = lax.axis_index("s") * NC + lax.axis_index("c")
    base = wid * NB
    pltpu.sync_copy(tokens_hbm.at[pl.ds(base, NB)], tok_v)

    def fire(b, buf, sem):
        pltpu.async_copy(table_hbm.at[tok_v.at[b, pl.ds(0, C0)]],
                         buf.at[pl.ds(0, C0)], sem)
        pltpu.async_copy(table_hbm.at[tok_v.at[b, pl.ds(C0, C1)]],
                         buf.at[pl.ds(C0, C1)], sem)

    def wait(buf, sem):
        pltpu.make_async_copy(table_hbm.at[pl.ds(0, L)], buf, sem).wait()

    zeros = jnp.zeros((16,), jnp.float32)

    def process(b, buf):
        def acc_body(i8, accs):
            t0 = i8 * 8
            for dt in range(8):
                accs = tuple(a + buf[t0 + dt, pl.ds(k * 16, 16)]
                             for k, a in enumerate(accs))
            return accs

        accs = lax.fori_loop(0, L // 8, acc_body, (zeros, zeros, zeros, zeros))
        for k in range(4):
            out_v[pl.ds(b * D + k * 16, 16)] = accs[k]

    fire(0, buf0, sem0)
    fire(1, buf1, sem1)

    def loop_body(i, carry):
        b0 = 2 * i
        wait(buf0, sem0)
        process(b0, buf0)

        @pl.when(i < NB // 2 - 1)
        def _():
            fire(b0 + 2, buf0, sem0)

        wait(buf1, sem1)
        process(b0 + 1, buf1)

        @pl.when(i < NB // 2 - 1)
        def _():
            fire(b0 + 3, buf1, sem1)

        return carry

    lax.fori_loop(0, NB // 2, loop_body, 0)
    pltpu.sync_copy(out_v, pooled_hbm.at[pl.ds(base * D, NB * D)])


_pool = functools.partial(
    pl.kernel,
    mesh=plsc.VectorSubcoreMesh(core_axis_name="c", subcore_axis_name="s"),
    compiler_params=pltpu.CompilerParams(use_tc_tiling_on_sc=False),
    out_type=jax.ShapeDtypeStruct((B * D,), jnp.float32),
    scratch_types=[
        pltpu.VMEM((NB, L), jnp.int32),
        pltpu.VMEM((L, D), jnp.float32),
        pltpu.VMEM((L, D), jnp.float32),
        pltpu.VMEM((NB * D,), jnp.float32),
        pltpu.SemaphoreType.DMA,
        pltpu.SemaphoreType.DMA,
    ],
)(_pool_body)


def _mlp_body(x_ref, tok_ref, w1_ref, b1_ref, w2_ref, b2_ref, o_ref):
    cnt = jnp.sum((tok_ref[...] != 0).astype(jnp.float32), axis=1,
                  keepdims=True)
    x = x_ref[...] / jnp.maximum(cnt, 1.0)
    h = jnp.dot(x, w1_ref[...], preferred_element_type=jnp.float32)
    h = jnp.maximum(h + b1_ref[...], 0.0)
    p = jnp.dot(h, w2_ref[...], preferred_element_type=jnp.float32)
    p = p + b2_ref[...]
    norm = jnp.sqrt(jnp.sum(p * p, axis=-1, keepdims=True))
    o_ref[...] = p / jnp.maximum(norm, 1e-8)


BLK = 512


def _mlp(summed, tokens, W1, b1, W2, b2):
    return pl.pallas_call(
        _mlp_body,
        out_shape=jax.ShapeDtypeStruct((B, O), jnp.float32),
        grid=(B // BLK,),
        in_specs=[
            pl.BlockSpec((BLK, D), lambda i: (i, 0)),
            pl.BlockSpec((BLK, L), lambda i: (i, 0)),
            pl.BlockSpec((D, O), lambda i: (0, 0)),
            pl.BlockSpec((1, O), lambda i: (0, 0)),
            pl.BlockSpec((O, O), lambda i: (0, 0)),
            pl.BlockSpec((1, O), lambda i: (0, 0)),
        ],
        out_specs=pl.BlockSpec((BLK, O), lambda i: (i, 0)),
    )(summed, tokens, W1, b1, W2, b2)


def kernel(tokens, table, W1, b1, W2, b2):
    summed = _pool(tokens, table).reshape(B, D)
    return _mlp(summed, tokens, W1, b1.reshape(1, O), W2, b2.reshape(1, O))
